# Initial kernel scaffold; baseline (speedup 1.0000x reference)
import jax
import jax.numpy as jnp
from jax.experimental import pallas as pl


def kernel(seqs, cell_emb, emb_table, W, b, gamma, beta):
    raise NotImplementedError("write your pallas kernel here")


# speedup vs baseline: 2.7447x; 2.7447x over previous
import functools
import math

import numpy as np
import jax
import jax.numpy as jnp
from jax import lax
from jax.experimental import pallas as pl
from jax.experimental.pallas import tpu as pltpu
from jax.experimental.pallas import tpu_sc as plsc

VOCAB = 100000
EMB = 64
CELL_IN = 128
BATCH = 4096
L = 200


def _make_pe_np():
    position = np.arange(0, L, dtype=np.float32)[:, None]
    div_term = np.exp(
        np.arange(0, EMB, 2, dtype=np.float32) * (-math.log(10000.0) / EMB))
    pe = np.zeros((L, EMB), dtype=np.float32)
    pe[:, 0::2] = np.sin(position * div_term)
    pe[:, 1::2] = np.cos(position * div_term)
    return pe


_PE = _make_pe_np()


def _cell_proj_tc(cell_emb, W, b):
    BB = 512

    def body(x_ref, w_ref, b_ref, o_ref):
        o_ref[...] = (
            jnp.dot(x_ref[...], w_ref[...].T, preferred_element_type=jnp.float32)
            + b_ref[...])

    return pl.pallas_call(
        body,
        grid=(BATCH // BB,),
        in_specs=[
            pl.BlockSpec((BB, CELL_IN), lambda i: (i, 0)),
            pl.BlockSpec((EMB, CELL_IN), lambda i: (0, 0)),
            pl.BlockSpec((1, EMB), lambda i: (0, 0)),
        ],
        out_specs=pl.BlockSpec((BB, EMB), lambda i: (i, 0)),
        out_shape=jax.ShapeDtypeStruct((BATCH, EMB), jnp.float32),
    )(cell_emb, W, b.reshape(1, EMB))


def _sc_embed(seqs, emb_table, cell_proj, gamma, beta, pe):
    info = plsc.get_sparse_core_info()
    NC, NS = info.num_cores, info.num_subcores
    NW = NC * NS
    BPW = BATCH // NW

    mesh = plsc.VectorSubcoreMesh(core_axis_name="c", subcore_axis_name="s")

    @functools.partial(
        pl.kernel,
        mesh=mesh,
        compiler_params=pltpu.CompilerParams(
            needs_layout_passes=False, use_tc_tiling_on_sc=False),
        out_type=jax.ShapeDtypeStruct((BATCH * L, EMB), jnp.float32),
        scratch_types=[
            pltpu.VMEM((L,), jnp.int32),
            pltpu.VMEM((EMB,), jnp.float32),
            pltpu.VMEM((L, EMB), jnp.float32),
            pltpu.VMEM((EMB,), jnp.float32),
            pltpu.VMEM((EMB,), jnp.float32),
            pltpu.VMEM((L, EMB), jnp.float32),
            pltpu.VMEM((L, EMB), jnp.float32),
            pltpu.SemaphoreType.DMA,
        ],
    )
    def k(seqs_hbm, table_hbm, cell_hbm, gamma_hbm, beta_hbm, pe_hbm,
          out_hbm, idx_v, cell_v, pe_v, g_v, be_v, rows_v, out_v, gsem):
        wid = lax.axis_index("s") * NC + lax.axis_index("c")
        b0 = wid * BPW
        pltpu.sync_copy(pe_hbm, pe_v)
        pltpu.sync_copy(gamma_hbm, g_v)
        pltpu.sync_copy(beta_hbm, be_v)

        gj = [g_v[pl.ds(16 * j, 16)] for j in range(4)]
        bj = [be_v[pl.ds(16 * j, 16)] for j in range(4)]

        def bbody(bi, _):
            bb = b0 + bi
            pltpu.sync_copy(seqs_hbm.at[bb], idx_v)
            pltpu.sync_copy(cell_hbm.at[bb], cell_v)
            c1 = pltpu.async_copy(
                table_hbm.at[idx_v.at[pl.ds(0, 120)]],
                rows_v.at[pl.ds(0, 120)], gsem)
            c2 = pltpu.async_copy(
                table_hbm.at[idx_v.at[pl.ds(120, 80)]],
                rows_v.at[pl.ds(120, 80)], gsem)
            c1.wait()
            c2.wait()

            cj = [cell_v[pl.ds(16 * j, 16)] for j in range(4)]

            def rowbody(p, _):
                t = [rows_v[p, pl.ds(16 * j, 16)] + cj[j]
                     + pe_v[p, pl.ds(16 * j, 16)] for j in range(4)]
                s = (t[0] + t[1]) + (t[2] + t[3])
                q = (t[0] * t[0] + t[1] * t[1]) + (t[2] * t[2] + t[3] * t[3])
                mu = jnp.sum(s) * (1.0 / 64.0)
                ex2 = jnp.sum(q) * (1.0 / 64.0)
                v = (ex2 - mu * mu) + 1e-12
                i32 = lax.bitcast_convert_type(v, jnp.int32)
                i32 = jnp.int32(0x5F3759DF) - lax.shift_right_logical(i32, 1)
                y = lax.bitcast_convert_type(i32, jnp.float32)
                h = 0.5 * v
                y = y * (1.5 - h * y * y)
                y = y * (1.5 - h * y * y)
                y = y * (1.5 - h * y * y)
                for j in range(4):
                    out_v[p, pl.ds(16 * j, 16)] = (t[j] - mu) * (gj[j] * y) + bj[j]
                return 0

            lax.fori_loop(0, L, rowbody, 0)
            pltpu.sync_copy(out_v, out_hbm.at[pl.ds(pl.multiple_of(bb * L, 8), L)])
            return 0

        lax.fori_loop(0, BPW, bbody, 0)

    return k(seqs, emb_table, cell_proj, gamma, beta, pe)


def kernel(seqs, cell_emb, emb_table, W, b, gamma, beta):
    cell_proj = _cell_proj_tc(cell_emb, W, b)
    pe = jnp.asarray(_PE)
    out = _sc_embed(seqs.astype(jnp.int32), emb_table, cell_proj, gamma, beta,
                    pe)
    return (out.reshape(BATCH, L, EMB), cell_proj)


# speedup vs baseline: 4.4065x; 1.6055x over previous
import functools
import math

import numpy as np
import jax
import jax.numpy as jnp
from jax import lax
from jax.experimental import pallas as pl
from jax.experimental.pallas import tpu as pltpu
from jax.experimental.pallas import tpu_sc as plsc

VOCAB = 100000
EMB = 64
CELL_IN = 128
BATCH = 4096
L = 200


def _make_pe_np():
    position = np.arange(0, L, dtype=np.float32)[:, None]
    div_term = np.exp(
        np.arange(0, EMB, 2, dtype=np.float32) * (-math.log(10000.0) / EMB))
    pe = np.zeros((L, EMB), dtype=np.float32)
    pe[:, 0::2] = np.sin(position * div_term)
    pe[:, 1::2] = np.cos(position * div_term)
    return pe


_PE = _make_pe_np()


def _cell_proj_tc(cell_emb, W, b):
    BB = 512

    def body(x_ref, w_ref, b_ref, o_ref):
        o_ref[...] = (
            jnp.dot(x_ref[...], w_ref[...].T, preferred_element_type=jnp.float32)
            + b_ref[...])

    return pl.pallas_call(
        body,
        grid=(BATCH // BB,),
        in_specs=[
            pl.BlockSpec((BB, CELL_IN), lambda i: (i, 0)),
            pl.BlockSpec((EMB, CELL_IN), lambda i: (0, 0)),
            pl.BlockSpec((1, EMB), lambda i: (0, 0)),
        ],
        out_specs=pl.BlockSpec((BB, EMB), lambda i: (i, 0)),
        out_shape=jax.ShapeDtypeStruct((BATCH, EMB), jnp.float32),
    )(cell_emb, W, b.reshape(1, EMB))


def _sc_embed(seqs_flat, emb_table, cell_flat, gamma, beta, pe_flat):
    info = plsc.get_sparse_core_info()
    NC, NS = info.num_cores, info.num_subcores
    NW = NC * NS
    BPW = BATCH // NW

    mesh = plsc.VectorSubcoreMesh(core_axis_name="c", subcore_axis_name="s")

    @functools.partial(
        pl.kernel,
        mesh=mesh,
        compiler_params=pltpu.CompilerParams(
            needs_layout_passes=False, use_tc_tiling_on_sc=False),
        out_type=jax.ShapeDtypeStruct((BATCH * L, EMB), jnp.float32),
        scratch_types=[
            pltpu.VMEM((BPW * L,), jnp.int32),
            pltpu.VMEM((BPW * EMB,), jnp.float32),
            pltpu.VMEM((L * EMB,), jnp.float32),
            pltpu.VMEM((EMB,), jnp.float32),
            pltpu.VMEM((EMB,), jnp.float32),
            pltpu.VMEM((2, L, EMB), jnp.float32),
            pltpu.VMEM((2, L, EMB), jnp.float32),
            [pltpu.SemaphoreType.DMA] * 2,
            [pltpu.SemaphoreType.DMA] * 2,
        ],
    )
    def k(seqs_hbm, table_hbm, cell_hbm, gamma_hbm, beta_hbm, pe_hbm,
          out_hbm, seqs_v, cell_v, pe_v, g_v, be_v, rows_v, out_v,
          gsem, osem):
        wid = lax.axis_index("s") * NC + lax.axis_index("c")
        b0 = wid * BPW
        pltpu.sync_copy(seqs_hbm.at[pl.ds(b0 * L, BPW * L)], seqs_v)
        pltpu.sync_copy(cell_hbm.at[pl.ds(b0 * EMB, BPW * EMB)], cell_v)
        pltpu.sync_copy(pe_hbm.at[pl.ds(0, L * EMB)], pe_v)
        pltpu.sync_copy(gamma_hbm, g_v)
        pltpu.sync_copy(beta_hbm, be_v)

        gj = [g_v[pl.ds(16 * j, 16)] for j in range(4)]
        bj = [be_v[pl.ds(16 * j, 16)] for j in range(4)]

        def gather_copies(bi, s):
            return (
                pltpu.make_async_copy(
                    table_hbm.at[seqs_v.at[pl.ds(bi * L, 120)]],
                    rows_v.at[s, pl.ds(0, 120)], gsem[s]),
                pltpu.make_async_copy(
                    table_hbm.at[seqs_v.at[pl.ds(bi * L + 120, 80)]],
                    rows_v.at[s, pl.ds(120, 80)], gsem[s]),
            )

        def out_copy(bi, s):
            return pltpu.make_async_copy(
                out_v.at[s],
                out_hbm.at[pl.ds(pl.multiple_of((b0 + bi) * L, 8), L)],
                osem[s])

        def issue_gather(bi, s):
            for c in gather_copies(bi, s):
                c.start()

        def compute(bi, s):
            @functools.partial(plsc.parallel_loop, 0, L, unroll=4)
            def rowbody(p):
                cj0 = cell_v[pl.ds(bi * EMB, 16)]
                cj1 = cell_v[pl.ds(bi * EMB + 16, 16)]
                cj2 = cell_v[pl.ds(bi * EMB + 32, 16)]
                cj3 = cell_v[pl.ds(bi * EMB + 48, 16)]
                t = [rows_v[s, p, pl.ds(16 * j, 16)] + cj
                     for j, cj in enumerate((cj0, cj1, cj2, cj3))]
                t = [t[j] + pe_v[pl.ds(p * EMB + 16 * j, 16)] for j in range(4)]
                ssum = (t[0] + t[1]) + (t[2] + t[3])
                q = (t[0] * t[0] + t[1] * t[1]) + (t[2] * t[2] + t[3] * t[3])
                mu = jnp.sum(ssum) * (1.0 / 64.0)
                ex2 = jnp.sum(q) * (1.0 / 64.0)
                v = (ex2 - mu * mu) + 1e-12
                i32 = lax.bitcast_convert_type(v, jnp.int32)
                i32 = jnp.int32(0x5F3759DF) - lax.shift_right_logical(i32, 1)
                y = lax.bitcast_convert_type(i32, jnp.float32)
                h = 0.5 * v
                y = y * (1.5 - h * y * y)
                y = y * (1.5 - h * y * y)
                y = y * (1.5 - h * y * y)
                for j in range(4):
                    out_v[s, p, pl.ds(16 * j, 16)] = (
                        (t[j] - mu) * (gj[j] * y) + bj[j])

        issue_gather(0, 0)

        def pair_body(g, _):
            for s in range(2):
                bi = 2 * g + s
                ns = 1 - s

                @pl.when(bi + 1 < BPW)
                def _():
                    issue_gather(bi + 1, ns)

                @pl.when(bi >= 2)
                def _():
                    out_copy(bi - 2, s).wait()

                for c in gather_copies(bi, s):
                    c.wait()
                compute(bi, s)
                out_copy(bi, s).start()
            return 0

        lax.fori_loop(0, BPW // 2, pair_body, 0)
        out_copy(BPW - 2, 0).wait()
        out_copy(BPW - 1, 1).wait()

    return k(seqs_flat, emb_table, cell_flat, gamma, beta, pe_flat)


def kernel(seqs, cell_emb, emb_table, W, b, gamma, beta):
    cell_proj = _cell_proj_tc(cell_emb, W, b)
    pe = jnp.asarray(_PE.reshape(-1))
    out = _sc_embed(seqs.astype(jnp.int32).reshape(-1), emb_table,
                    cell_proj.reshape(-1), gamma, beta, pe)
    return (out.reshape(BATCH, L, EMB), cell_proj)
